# Initial kernel scaffold; baseline (speedup 1.0000x reference)
import jax
import jax.numpy as jnp
from jax.experimental import pallas as pl


def kernel(seq_indices, legal_mask):
    raise NotImplementedError("write your pallas kernel here")


# speedup vs baseline: 212.1101x; 212.1101x over previous
import jax
import jax.numpy as jnp
from jax import lax
from jax.experimental import pallas as pl
from jax.experimental.pallas import tpu as pltpu
from jax.experimental.pallas import tpu_sc as plsc

B, L, CH, NCH = 4, 256, 96, 6
PADLEN = 272
NW = 32
ROWS_PER_W = (B * L) // NW
WORKERS_PER_B = NW // B


def _sc_body(padded_hbm, legal_hbm, out_hbm,
             padbuf, acoltab, bjtab, legalbuf, outbuf):
    wid = lax.axis_index("s") * 2 + lax.axis_index("c")
    b = wid // WORKERS_PER_B
    i0 = (wid % WORKERS_PER_B) * ROWS_PER_W

    lv = lax.iota(jnp.int32, 16)
    lm8 = lv < 8
    selrow = (lv & 7) < 4
    colsel = (lv & 7) >= 4
    c4 = lv & 3
    one = jnp.full((16,), 1.0, jnp.float32)
    zero = jnp.zeros((16,), jnp.float32)

    offi = [jnp.where(lm8, 5, 0) if c == 0
            else jnp.where(lm8, 2 * c - 1, 2 * c) for c in range(NCH)]
    offj = [jnp.where(lm8, 5, 10) if c == 0
            else jnp.where(lm8, 11 - 2 * c, 10 - 2 * c) for c in range(NCH)]
    koff = [jnp.where(lm8, -1000, -7) if c == 0
            else jnp.where(lm8, 4 * c - 9, 4 * c - 7) for c in range(NCH)]

    pltpu.sync_copy(padded_hbm.at[b], padbuf)
    pltpu.sync_copy(legal_hbm.at[b, pl.ds(i0, ROWS_PER_W)],
                    legalbuf.at[:, pl.ds(0, L)])

    def build_tables(j, carry):
        jsplat = jnp.full((16,), j, jnp.int32)
        for c in range(NCH):
            bj = plsc.load_gather(padbuf, [jsplat + offj[c]])
            acol = jnp.where(colsel, (bj == c4).astype(jnp.float32), one)
            bjtab[j, pl.ds(16 * c, 16)] = bj
            acoltab[j, pl.ds(16 * c, 16)] = acol
        return carry

    lax.fori_loop(0, L, build_tables, 0)

    def row_body(r, carry):
        i = i0 + r
        isplat = jnp.full((16,), i, jnp.int32)
        bis, arows, thrs = [], [], []
        for c in range(NCH):
            bi = plsc.load_gather(padbuf, [isplat + offi[c]])
            bis.append(bi)
            arows.append(jnp.where(selrow, (bi == c4).astype(jnp.float32), one))
            thrs.append(isplat + koff[c])

        def col_body(j, inner):
            lg = legalbuf[r, pl.ds(j, 16)][0]
            lgv = jnp.full((16,), lg, jnp.float32)
            lm0 = jnp.where(lm8, one, lgv)
            jv = jnp.full((16,), j, jnp.int32)
            for c in range(NCH):
                acol = acoltab[j, pl.ds(16 * c, 16)]
                bj = bjtab[j, pl.ds(16 * c, 16)]
                su = bis[c] + bj
                m = (su == 3) | (su == 5)
                if c == 0:
                    m = m | lm8
                m = m & (jv > thrs[c])
                lgc = lm0 if c == 0 else lgv
                outbuf[j, pl.ds(16 * c, 16)] = jnp.where(m, arows[c] * lgc * acol, zero)
            return inner

        lax.fori_loop(0, L, col_body, 0)
        pltpu.sync_copy(outbuf, out_hbm.at[b, i])
        return carry

    lax.fori_loop(0, ROWS_PER_W, row_body, 0)


def kernel(seq_indices, legal_mask):
    padded = jnp.pad(seq_indices.astype(jnp.int32), ((0, 0), (5, 11)),
                     constant_values=1000)
    padded = jnp.where(padded >= 4, 1000, padded)
    f = pl.kernel(
        _sc_body,
        out_type=jax.ShapeDtypeStruct((B, L, L, CH), jnp.float32),
        mesh=plsc.VectorSubcoreMesh(core_axis_name="c", subcore_axis_name="s"),
        compiler_params=pltpu.CompilerParams(needs_layout_passes=False),
        scratch_types=[
            pltpu.VMEM((PADLEN,), jnp.int32),
            pltpu.VMEM((L, CH), jnp.float32),
            pltpu.VMEM((L, CH), jnp.int32),
            pltpu.VMEM((ROWS_PER_W, PADLEN), jnp.float32),
            pltpu.VMEM((L, CH), jnp.float32),
        ],
    )
    return f(padded, legal_mask)


# speedup vs baseline: 256.3394x; 1.2085x over previous
import jax
import jax.numpy as jnp
from jax import lax
from jax.experimental import pallas as pl
from jax.experimental.pallas import tpu as pltpu
from jax.experimental.pallas import tpu_sc as plsc

B, L, CH, NCH = 4, 256, 96, 6
PADLEN = 272
NW = 32
ROWS_PER_W = (B * L) // NW
WORKERS_PER_B = NW // B


def _sc_body(padded_hbm, legal_hbm, out_hbm,
             padbuf, colmtab, legalbuf, outbuf, sem):
    wid = lax.axis_index("s") * 2 + lax.axis_index("c")
    b = wid // WORKERS_PER_B
    i0 = (wid % WORKERS_PER_B) * ROWS_PER_W

    lv = lax.iota(jnp.int32, 16)
    lm8 = lv < 8
    selrow = (lv & 7) < 4
    c4 = lv & 3
    t3 = 3 - c4
    t5 = 5 - c4
    one = jnp.full((16,), 1.0, jnp.float32)
    zero = jnp.zeros((16,), jnp.float32)

    offi = [jnp.where(lm8, 5, 0) if c == 0
            else jnp.where(lm8, 2 * c - 1, 2 * c) for c in range(NCH)]
    offj = [jnp.where(lm8, 5, 10) if c == 0
            else jnp.where(lm8, 11 - 2 * c, 10 - 2 * c) for c in range(NCH)]
    koff = [jnp.where(lm8, -1000, -7) if c == 0
            else jnp.where(lm8, 4 * c - 9, 4 * c - 7) for c in range(NCH)]

    pltpu.sync_copy(padded_hbm.at[b], padbuf)
    pltpu.sync_copy(legal_hbm.at[b, pl.ds(i0, ROWS_PER_W)],
                    legalbuf.at[:, pl.ds(0, L)])

    def build_tables(j, carry):
        jsplat = jnp.full((16,), j, jnp.int32)
        for c in range(NCH):
            bj = plsc.load_gather(padbuf, [jsplat + offj[c]])
            v = jnp.where(selrow, (bj == t3) | (bj == t5), bj == c4)
            vf = v.astype(jnp.float32)
            if c == 0:
                vf = jnp.where(lv < 4, one, vf)
            colmtab[j, pl.ds(16 * c, 16)] = vf
        return carry

    lax.fori_loop(0, L, build_tables, 0)

    def row_body(r, carry):
        i = i0 + r
        slot = lax.rem(r, 2)
        isplat = jnp.full((16,), i, jnp.int32)
        rowms, thrs = [], []
        for c in range(NCH):
            bi = plsc.load_gather(padbuf, [isplat + offi[c]])
            v = jnp.where(selrow, bi == c4, (bi == t3) | (bi == t5))
            vf = v.astype(jnp.float32)
            if c == 0:
                vf = jnp.where((lv >= 4) & lm8, one, vf)
            rowms.append(vf)
            thrs.append(isplat + koff[c])

        @pl.when(r >= 2)
        def _drain():
            pltpu.make_async_copy(outbuf.at[slot], out_hbm.at[b, i], sem).wait()

        obuf = outbuf.at[slot]

        def col_body(j, inner):
            lg = legalbuf[r, pl.ds(j, 16)][0]
            lgv = jnp.full((16,), lg, jnp.float32)
            lm0 = jnp.where(lm8, one, lgv)
            jv = jnp.full((16,), j, jnp.int32)
            for c in range(NCH):
                cm = colmtab[j, pl.ds(16 * c, 16)]
                rowlg = rowms[c] * (lm0 if c == 0 else lgv)
                obuf[j, pl.ds(16 * c, 16)] = jnp.where(
                    jv > thrs[c], rowlg * cm, zero)
            return inner

        lax.fori_loop(0, L, col_body, 0, unroll=4)
        pltpu.async_copy(outbuf.at[slot], out_hbm.at[b, i], sem)
        return carry

    lax.fori_loop(0, ROWS_PER_W, row_body, 0)
    pltpu.make_async_copy(outbuf.at[0], out_hbm.at[b, i0], sem).wait()
    pltpu.make_async_copy(outbuf.at[1], out_hbm.at[b, i0], sem).wait()


def kernel(seq_indices, legal_mask):
    padded = jnp.pad(seq_indices.astype(jnp.int32), ((0, 0), (5, 11)),
                     constant_values=1000)
    padded = jnp.where(padded >= 4, 1000, padded)
    f = pl.kernel(
        _sc_body,
        out_type=jax.ShapeDtypeStruct((B, L, L, CH), jnp.float32),
        mesh=plsc.VectorSubcoreMesh(core_axis_name="c", subcore_axis_name="s"),
        compiler_params=pltpu.CompilerParams(needs_layout_passes=False),
        scratch_types=[
            pltpu.VMEM((PADLEN,), jnp.int32),
            pltpu.VMEM((L, CH), jnp.float32),
            pltpu.VMEM((ROWS_PER_W, PADLEN), jnp.float32),
            pltpu.VMEM((2, L, CH), jnp.float32),
            pltpu.SemaphoreType.DMA,
        ],
    )
    return f(padded, legal_mask)


# speedup vs baseline: 566.0709x; 2.2083x over previous
import jax
import jax.numpy as jnp
from jax import lax
from jax.experimental import pallas as pl
from jax.experimental.pallas import tpu as pltpu
from jax.experimental.pallas import tpu_sc as plsc

B, L, CH, NCH = 4, 256, 96, 6
PADLEN = 272
NW = 32
ROWS_PER_W = (B * L) // NW
WORKERS_PER_B = NW // B


def _sc_body(padded_hbm, legal_hbm, out_hbm,
             padbuf, colmtab, legalbuf, outbuf, sem0, sem1):
    wid = lax.axis_index("s") * 2 + lax.axis_index("c")
    b = wid // WORKERS_PER_B
    i0 = (wid % WORKERS_PER_B) * ROWS_PER_W

    lv = lax.iota(jnp.int32, 16)
    lm8 = lv < 8
    selrow = (lv & 7) < 4
    c4 = lv & 3
    t3 = 3 - c4
    t5 = 5 - c4
    one = jnp.full((16,), 1.0, jnp.float32)
    zero = jnp.zeros((16,), jnp.float32)

    offi = [jnp.where(lm8, 5, 0) if c == 0
            else jnp.where(lm8, 2 * c - 1, 2 * c) for c in range(NCH)]
    offj = [jnp.where(lm8, 5, 10) if c == 0
            else jnp.where(lm8, 11 - 2 * c, 10 - 2 * c) for c in range(NCH)]
    koff = [jnp.where(lm8, -1000, -7) if c == 0
            else jnp.where(lm8, 4 * c - 9, 4 * c - 7) for c in range(NCH)]

    pltpu.sync_copy(padded_hbm.at[b], padbuf)
    pltpu.sync_copy(legal_hbm.at[b, pl.ds(i0, ROWS_PER_W)],
                    legalbuf.at[:, pl.ds(0, L)])

    def build_tables(j, carry):
        jsplat = jnp.full((16,), j, jnp.int32)
        for c in range(NCH):
            bj = plsc.load_gather(padbuf, [jsplat + offj[c]])
            v = jnp.where(selrow, (bj == t3) | (bj == t5), bj == c4)
            vf = v.astype(jnp.float32)
            if c == 0:
                vf = jnp.where(lv < 4, one, vf)
            colmtab[j, pl.ds(16 * c, 16)] = vf
        return carry

    lax.fori_loop(0, L, build_tables, 0)

    def one_row(r, slot, sem):
        i = i0 + r
        isplat = jnp.full((16,), i, jnp.int32)
        rowms, thrs = [], []
        for c in range(NCH):
            bi = plsc.load_gather(padbuf, [isplat + offi[c]])
            v = jnp.where(selrow, bi == c4, (bi == t3) | (bi == t5))
            vf = v.astype(jnp.float32)
            if c == 0:
                vf = jnp.where((lv >= 4) & lm8, one, vf)
            rowms.append(vf)
            thrs.append(isplat + koff[c])

        @pl.when(r >= 2)
        def _drain():
            pltpu.make_async_copy(outbuf.at[slot], out_hbm.at[b, i], sem).wait()

        obuf = outbuf.at[slot]

        def grp_body(g, inner):
            j0 = g * 16
            lgv16 = legalbuf[r, pl.ds(j0, 16)]
            all_masked = (j0 + 15) < (i - 6)
            all_open = j0 > (i + 13)

            @pl.when(all_masked)
            def _a():
                for jj in range(16):
                    j = j0 + jj
                    cm0 = colmtab[j, pl.ds(0, 16)]
                    obuf[j, pl.ds(0, 16)] = jnp.where(lm8, rowms[0] * cm0, zero)
                    for c in range(1, NCH):
                        obuf[j, pl.ds(16 * c, 16)] = zero

            @pl.when(all_open)
            def _c():
                for jj in range(16):
                    j = j0 + jj
                    lgv = jnp.full((16,), lgv16[jj], jnp.float32)
                    lm0 = jnp.where(lm8, one, lgv)
                    for c in range(NCH):
                        cm = colmtab[j, pl.ds(16 * c, 16)]
                        obuf[j, pl.ds(16 * c, 16)] = (
                            rowms[c] * (lm0 if c == 0 else lgv)) * cm

            @pl.when(jnp.logical_not(all_masked | all_open))
            def _b():
                for jj in range(16):
                    j = j0 + jj
                    lgv = jnp.full((16,), lgv16[jj], jnp.float32)
                    lm0 = jnp.where(lm8, one, lgv)
                    jv = jnp.full((16,), j, jnp.int32)
                    for c in range(NCH):
                        cm = colmtab[j, pl.ds(16 * c, 16)]
                        rowlg = rowms[c] * (lm0 if c == 0 else lgv)
                        obuf[j, pl.ds(16 * c, 16)] = jnp.where(
                            jv > thrs[c], rowlg * cm, zero)
            return inner

        lax.fori_loop(0, 16, grp_body, 0)
        pltpu.async_copy(outbuf.at[slot], out_hbm.at[b, i], sem)

    def pair_body(p, carry):
        one_row(2 * p, 0, sem0)
        one_row(2 * p + 1, 1, sem1)
        return carry

    lax.fori_loop(0, ROWS_PER_W // 2, pair_body, 0)
    pltpu.make_async_copy(outbuf.at[0], out_hbm.at[b, i0], sem0).wait()
    pltpu.make_async_copy(outbuf.at[1], out_hbm.at[b, i0], sem1).wait()


def kernel(seq_indices, legal_mask):
    padded = jnp.pad(seq_indices.astype(jnp.int32), ((0, 0), (5, 11)),
                     constant_values=1000)
    padded = jnp.where(padded >= 4, 1000, padded)
    f = pl.kernel(
        _sc_body,
        out_type=jax.ShapeDtypeStruct((B, L, L, CH), jnp.float32),
        mesh=plsc.VectorSubcoreMesh(core_axis_name="c", subcore_axis_name="s"),
        compiler_params=pltpu.CompilerParams(needs_layout_passes=False),
        scratch_types=[
            pltpu.VMEM((PADLEN,), jnp.int32),
            pltpu.VMEM((L, CH), jnp.float32),
            pltpu.VMEM((ROWS_PER_W, PADLEN), jnp.float32),
            pltpu.VMEM((2, L, CH), jnp.float32),
            pltpu.SemaphoreType.DMA,
            pltpu.SemaphoreType.DMA,
        ],
    )
    return f(padded, legal_mask)
